# baseline (speedup 1.0000x reference)
import functools

import jax
import jax.numpy as jnp
from jax import lax
from jax.experimental import pallas as pl
from jax.experimental.pallas import tpu as pltpu
from jax.experimental.pallas import tpu_sc as plsc

H = 256
V = 29
L = 12
LANES = 16

NC, NS = 2, 16
NW = NC * NS
NB = 16384
B_TOTAL = NB * L
B_PER_W = B_TOTAL // NW
CHUNK = 64
N_CHUNKS = B_PER_W // CHUNK
NBUF = 4


def _fuse_body(word_ref, pos_ref, out_ref):
    out_ref[...] = (
        word_ref[...][None, :, None, :] + pos_ref[...][None, None, :, :]
    ).astype(jnp.bfloat16)


def _build_fused(word_table, pos_table):
    fused = pl.pallas_call(
        _fuse_body,
        grid=(NW,),
        in_specs=[
            pl.BlockSpec((V, H), lambda r: (0, 0)),
            pl.BlockSpec((L, H), lambda r: (0, 0)),
        ],
        out_specs=pl.BlockSpec((1, V, L, H), lambda r: (r, 0, 0, 0)),
        out_shape=jax.ShapeDtypeStruct((NW, V, L, H), jnp.bfloat16),
    )(word_table, pos_table)
    return fused.reshape(NW * V * L, H)


_sc_mesh = plsc.VectorSubcoreMesh(core_axis_name="c", subcore_axis_name="s")


@functools.partial(
    pl.kernel,
    out_type=jax.ShapeDtypeStruct((B_TOTAL, H), jnp.float32),
    mesh=_sc_mesh,
    scratch_types=[
        pltpu.VMEM((B_PER_W,), jnp.int32),
        [pltpu.VMEM((CHUNK, H // 2), jnp.int32) for _ in range(NBUF)],
        [pltpu.VMEM((CHUNK, H), jnp.float32) for _ in range(NBUF)],
        [pltpu.SemaphoreType.DMA for _ in range(NBUF)],
        [pltpu.SemaphoreType.DMA for _ in range(NBUF)],
    ],
)
def _sc_lookup(x_hbm, fused_hbm, out_hbm, idx_v, bbufs, fbufs, gsems, ssems):
    wid = lax.axis_index("s") * NC + lax.axis_index("c")
    base = wid * B_PER_W
    tab = wid * (V * L)

    pltpu.sync_copy(x_hbm.at[pl.ds(base, B_PER_W)], idx_v)

    def idx_body(j, carry):
        off = j * LANES
        x = idx_v[pl.ds(off, LANES)]
        pos = (base + off) // NB
        idx_v[pl.ds(off, LANES)] = x * L + (tab + pos)
        return carry

    lax.fori_loop(0, B_PER_W // LANES, idx_body, 0)

    def gather_start(g, b):
        pltpu.async_copy(
            fused_hbm.at[idx_v.at[pl.ds(g * CHUNK, CHUNK)]], bbufs[b], gsems[b]
        )

    def gather_wait(b):
        pltpu.make_async_copy(
            fused_hbm.at[pl.ds(0, CHUNK)], bbufs[b], gsems[b]
        ).wait()

    def widen(b):
        bbuf, fbuf = bbufs[b], fbufs[b]

        def row_body(r, carry):
            for k in range(H // 32):
                w = bbuf[r, pl.ds(k * 16, 16)]
                fbuf[r, pl.ds(k * 16, 16)] = lax.bitcast_convert_type(
                    w << 16, jnp.float32
                )
                fbuf[r, pl.ds(H // 2 + k * 16, 16)] = lax.bitcast_convert_type(
                    w & jnp.int32(-65536), jnp.float32
                )
            return carry

        lax.fori_loop(0, CHUNK, row_body, 0)

    def scatter_start(g, b):
        pltpu.async_copy(
            fbufs[b], out_hbm.at[pl.ds(base + g * CHUNK, CHUNK)], ssems[b]
        )

    def scatter_wait(g, b):
        pltpu.make_async_copy(
            fbufs[b], out_hbm.at[pl.ds(base + g * CHUNK, CHUNK)], ssems[b]
        ).wait()

    for b in range(NBUF):
        gather_start(b, b)

    def chunk_body(t, carry):
        g0 = t * NBUF
        for b in range(NBUF):
            gather_wait(b)
            widen(b)
            scatter_start(g0 + b, b)
        for b in range(NBUF):
            scatter_wait(g0 + b, b)
            gather_start(g0 + NBUF + b, b)
        return carry

    lax.fori_loop(0, N_CHUNKS // NBUF - 1, chunk_body, 0)

    g0 = N_CHUNKS - NBUF
    for b in range(NBUF):
        gather_wait(b)
        widen(b)
        scatter_start(g0 + b, b)
    for b in range(NBUF):
        scatter_wait(g0 + b, b)


def kernel(X, word_table, pos_table):
    fused = _build_fused(word_table, pos_table)
    fused_i32 = lax.bitcast_convert_type(
        jnp.stack([fused[:, : H // 2], fused[:, H // 2 :]], axis=-1),
        jnp.int32,
    )
    xt_flat = X.T.reshape(-1).astype(jnp.int32)
    out = _sc_lookup(xt_flat, fused_i32)
    return out.reshape(L, NB, H).transpose(1, 0, 2)


# speedup vs baseline: 1.4401x; 1.4401x over previous
import functools

import jax
import jax.numpy as jnp
from jax import lax
from jax.experimental import pallas as pl
from jax.experimental.pallas import tpu as pltpu
from jax.experimental.pallas import tpu_sc as plsc

H = 256
V = 29
L = 12
LANES = 16

NC, NS = 2, 16
NW = NC * NS
NB = 16384
B_TOTAL = NB * L
B_PER_W = B_TOTAL // NW
CHUNK = 96
N_CHUNKS = B_PER_W // CHUNK
NBUF = 4


def _fuse_body(word_ref, pos_ref, out_ref):
    out_ref[...] = (
        word_ref[...][None, :, None, :] + pos_ref[...][None, None, :, :]
    )


def _build_fused(word_table, pos_table):
    fused = pl.pallas_call(
        _fuse_body,
        grid=(NW,),
        in_specs=[
            pl.BlockSpec((V, H), lambda r: (0, 0)),
            pl.BlockSpec((L, H), lambda r: (0, 0)),
        ],
        out_specs=pl.BlockSpec((1, V, L, H), lambda r: (r, 0, 0, 0)),
        out_shape=jax.ShapeDtypeStruct((NW, V, L, H), jnp.float32),
    )(word_table, pos_table)
    return fused.reshape(NW * V * L, H)


_sc_mesh = plsc.VectorSubcoreMesh(core_axis_name="c", subcore_axis_name="s")


@functools.partial(
    pl.kernel,
    out_type=jax.ShapeDtypeStruct((B_TOTAL, H), jnp.float32),
    mesh=_sc_mesh,
    scratch_types=[
        pltpu.VMEM((B_PER_W,), jnp.int32),
        [pltpu.VMEM((CHUNK, H), jnp.float32) for _ in range(NBUF)],
        [pltpu.SemaphoreType.DMA for _ in range(NBUF)],
        [pltpu.SemaphoreType.DMA for _ in range(NBUF)],
    ],
)
def _sc_lookup(x_hbm, fused_hbm, out_hbm, idx_v, bufs, gsems, ssems):
    wid = lax.axis_index("s") * NC + lax.axis_index("c")
    base = wid * B_PER_W
    tab = wid * (V * L)

    pltpu.sync_copy(x_hbm.at[pl.ds(base, B_PER_W)], idx_v)

    def idx_body(j, carry):
        off = j * LANES
        x = idx_v[pl.ds(off, LANES)]
        pos = (base + off) // NB
        idx_v[pl.ds(off, LANES)] = x * L + (tab + pos)
        return carry

    lax.fori_loop(0, B_PER_W // LANES, idx_body, 0)

    def gather_start(g, b):
        pltpu.async_copy(
            fused_hbm.at[idx_v.at[pl.ds(g * CHUNK, CHUNK)]], bufs[b], gsems[b]
        )

    def gather_wait(b):
        pltpu.make_async_copy(out_hbm.at[pl.ds(0, CHUNK)], bufs[b], gsems[b]).wait()

    def scatter_start(g, b):
        pltpu.async_copy(bufs[b], out_hbm.at[pl.ds(base + g * CHUNK, CHUNK)], ssems[b])

    def scatter_wait(g, b):
        pltpu.make_async_copy(
            bufs[b], out_hbm.at[pl.ds(base + g * CHUNK, CHUNK)], ssems[b]
        ).wait()

    for b in range(NBUF):
        gather_start(b, b)

    def chunk_body(t, carry):
        g0 = t * NBUF
        for b in range(NBUF):
            gather_wait(b)
            scatter_start(g0 + b, b)
        for b in range(NBUF):
            scatter_wait(g0 + b, b)
            gather_start(g0 + NBUF + b, b)
        return carry

    lax.fori_loop(0, N_CHUNKS // NBUF - 1, chunk_body, 0)

    g0 = N_CHUNKS - NBUF
    for b in range(NBUF):
        gather_wait(b)
        scatter_start(g0 + b, b)
    for b in range(NBUF):
        scatter_wait(g0 + b, b)


def kernel(X, word_table, pos_table):
    fused = _build_fused(word_table, pos_table)
    xt_flat = X.T.reshape(-1).astype(jnp.int32)
    out = _sc_lookup(xt_flat, fused)
    return out.reshape(L, NB, H).transpose(1, 0, 2)
